# baseline (speedup 1.0000x reference)
import jax
import jax.numpy as jnp
from jax.experimental import pallas as pl

_N = 577
_ROW_BLOCK = 320

_RENORM_HEADS = (2, 3, 5, 7, 8)
_COPY_HEAD = 10


def _choose_attn_kernel(x_ref, o_ref):
    h = pl.program_id(1)
    x = x_ref[0, 0]

    is_renorm = (h == _RENORM_HEADS[0])
    for hh in _RENORM_HEADS[1:]:
        is_renorm = jnp.logical_or(is_renorm, h == hh)
    is_copy = h == _COPY_HEAD

    @pl.when(is_renorm)
    def _():
        col = jax.lax.broadcasted_iota(jnp.int32, x.shape, 1)
        t = jnp.where(col < _N, jnp.maximum(x, 0.0), 0.0)
        s = jnp.sum(t, axis=1, keepdims=True)
        o_ref[0, 0] = t / (s + 1e-5)

    @pl.when(is_copy)
    def _():
        o_ref[0, 0] = x

    @pl.when(jnp.logical_not(jnp.logical_or(is_renorm, is_copy)))
    def _():
        o_ref[0, 0] = x * (1.0 / _N)


def kernel(attn_weights):
    b, nh, n, _ = attn_weights.shape
    grid = (b, nh, pl.cdiv(n, _ROW_BLOCK))
    spec = pl.BlockSpec(
        (1, 1, _ROW_BLOCK, n), lambda i, j, k: (i, j, k, 0)
    )
    return pl.pallas_call(
        _choose_attn_kernel,
        grid=grid,
        in_specs=[spec],
        out_specs=spec,
        out_shape=jax.ShapeDtypeStruct(attn_weights.shape, attn_weights.dtype),
    )(attn_weights)


# speedup vs baseline: 1.2051x; 1.2051x over previous
import jax
import jax.numpy as jnp
from jax.experimental import pallas as pl

_N = 577
_ROW_BLOCK = 577

_RENORM_HEADS = (2, 3, 5, 7, 8)
_COPY_HEAD = 10


def _choose_attn_kernel(x_ref, o_ref):
    h = pl.program_id(1)
    x = x_ref[0, 0]

    is_renorm = (h == _RENORM_HEADS[0])
    for hh in _RENORM_HEADS[1:]:
        is_renorm = jnp.logical_or(is_renorm, h == hh)
    is_copy = h == _COPY_HEAD

    @pl.when(is_renorm)
    def _():
        col = jax.lax.broadcasted_iota(jnp.int32, x.shape, 1)
        t = jnp.where(col < _N, jnp.maximum(x, 0.0), 0.0)
        ones = jnp.ones((x.shape[1], 1), dtype=x.dtype)
        s = jax.lax.dot_general(
            t, ones, (((1,), (0,)), ((), ())),
            preferred_element_type=jnp.float32,
        )
        o_ref[0, 0] = t * (1.0 / (s + 1e-5))

    @pl.when(is_copy)
    def _():
        o_ref[0, 0] = x

    @pl.when(jnp.logical_not(jnp.logical_or(is_renorm, is_copy)))
    def _():
        o_ref[0, 0] = x * (1.0 / _N)


def kernel(attn_weights):
    b, nh, n, _ = attn_weights.shape
    grid = (b, nh, pl.cdiv(n, _ROW_BLOCK))
    spec = pl.BlockSpec(
        (1, 1, _ROW_BLOCK, n), lambda i, j, k: (i, j, k, 0)
    )
    return pl.pallas_call(
        _choose_attn_kernel,
        grid=grid,
        in_specs=[spec],
        out_specs=spec,
        out_shape=jax.ShapeDtypeStruct(attn_weights.shape, attn_weights.dtype),
    )(attn_weights)


# speedup vs baseline: 1.2331x; 1.0232x over previous
import jax
import jax.numpy as jnp
from jax.experimental import pallas as pl

_N = 577
_ROW_BLOCK = 577

_RENORM_HEADS = (2, 3, 5, 7, 8)
_COPY_HEAD = 10


def _choose_attn_kernel(x_ref, o_ref):
    h = pl.program_id(1)
    x = x_ref[0, 0]

    is_renorm = (h == _RENORM_HEADS[0])
    for hh in _RENORM_HEADS[1:]:
        is_renorm = jnp.logical_or(is_renorm, h == hh)
    is_copy = h == _COPY_HEAD

    @pl.when(is_renorm)
    def _():
        col = jax.lax.broadcasted_iota(jnp.int32, x.shape, 1)
        t = jnp.where(col < _N, jnp.maximum(x, 0.0), 0.0)
        s = jnp.sum(t, axis=1, keepdims=True)
        o_ref[0, 0] = t / (s + 1e-5)

    @pl.when(is_copy)
    def _():
        o_ref[0, 0] = x

    @pl.when(jnp.logical_not(jnp.logical_or(is_renorm, is_copy)))
    def _():
        o_ref[0, 0] = x * (1.0 / _N)


def kernel(attn_weights):
    b, nh, n, _ = attn_weights.shape
    grid = (b, nh, pl.cdiv(n, _ROW_BLOCK))
    spec = pl.BlockSpec(
        (1, 1, _ROW_BLOCK, n), lambda i, j, k: (i, j, k, 0)
    )
    return pl.pallas_call(
        _choose_attn_kernel,
        grid=grid,
        in_specs=[spec],
        out_specs=spec,
        out_shape=jax.ShapeDtypeStruct(attn_weights.shape, attn_weights.dtype),
    )(attn_weights)
